# baseline (speedup 1.0000x reference)
import functools

import jax
import jax.numpy as jnp
from jax import lax
from jax.experimental import pallas as pl
from jax.experimental.pallas import tpu as pltpu
from jax.experimental.pallas import tpu_sc as plsc

EMBEDDING_DIM = 64
_NUM_CORES = 2
_NUM_SUBCORES = 16
_NW = _NUM_CORES * _NUM_SUBCORES

_CHUNK = 128


def _build(total_rows: int):
    assert total_rows % (_NW * _CHUNK) == 0
    per_worker = total_rows // _NW
    n_chunks = per_worker // _CHUNK

    mesh = plsc.VectorSubcoreMesh(core_axis_name="c", subcore_axis_name="s")

    @functools.partial(
        pl.kernel,
        mesh=mesh,
        compiler_params=pltpu.CompilerParams(use_tc_tiling_on_sc=False),
        out_type=jax.ShapeDtypeStruct((total_rows, EMBEDDING_DIM), jnp.float32),
        scratch_types=[
            pltpu.VMEM((_CHUNK,), jnp.int32),
            pltpu.VMEM((_CHUNK, EMBEDDING_DIM), jnp.float32),
            pltpu.SemaphoreType.DMA,
        ],
    )
    def emb(idx_hbm, table_hbm, out_hbm, idx_v, rows_v, sem):
        wid = lax.axis_index("s") * _NUM_CORES + lax.axis_index("c")
        base = wid * per_worker

        def body(i, carry):
            off = base + i * _CHUNK
            pltpu.sync_copy(idx_hbm.at[pl.ds(off, _CHUNK)], idx_v)
            pltpu.async_copy(table_hbm.at[idx_v], rows_v, sem).wait()
            pltpu.sync_copy(rows_v, out_hbm.at[pl.ds(off, _CHUNK)])
            return carry

        lax.fori_loop(0, n_chunks, body, 0)

    return emb


def kernel(input, weight):
    batch, seq = input.shape
    idx = input.reshape(-1).astype(jnp.int32)
    out = _build(batch * seq)(idx, weight)
    return out.reshape(batch, seq, EMBEDDING_DIM)


# speedup vs baseline: 1.1954x; 1.1954x over previous
import functools

import jax
import jax.numpy as jnp
from jax import lax
from jax.experimental import pallas as pl
from jax.experimental.pallas import tpu as pltpu
from jax.experimental.pallas import tpu_sc as plsc

EMBEDDING_DIM = 64
_NUM_CORES = 2
_NUM_SUBCORES = 16
_NW = _NUM_CORES * _NUM_SUBCORES

_CHUNK = 128
_NBUF = 8


def _build(total_rows: int):
    assert total_rows % (_NW * _CHUNK) == 0
    per_worker = total_rows // _NW
    n_chunks = per_worker // _CHUNK
    assert n_chunks % _NBUF == 0
    n_groups = n_chunks // _NBUF

    mesh = plsc.VectorSubcoreMesh(core_axis_name="c", subcore_axis_name="s")

    @functools.partial(
        pl.kernel,
        mesh=mesh,
        compiler_params=pltpu.CompilerParams(use_tc_tiling_on_sc=False),
        out_type=jax.ShapeDtypeStruct((total_rows, EMBEDDING_DIM), jnp.float32),
        scratch_types=[
            pltpu.VMEM((n_chunks, _CHUNK), jnp.int32),
            pltpu.VMEM((_NBUF, _CHUNK, EMBEDDING_DIM), jnp.float32),
            pltpu.SemaphoreType.DMA((_NBUF,)),
        ],
    )
    def emb(idx_hbm, table_hbm, out_hbm, idx_v, rows_v, gsem):
        wid = lax.axis_index("s") * _NUM_CORES + lax.axis_index("c")
        base = wid * per_worker

        pltpu.sync_copy(idx_hbm.at[pl.ds(wid * n_chunks, n_chunks)], idx_v)

        def gather(chunk, b):
            pltpu.async_copy(
                table_hbm.at[idx_v.at[chunk]], rows_v.at[b], gsem.at[b]
            )

        for b in range(_NBUF):
            gather(b, b)

        def group_body(g, carry):
            for b in range(_NBUF):
                i = g * _NBUF + b
                pltpu.make_async_copy(
                    table_hbm.at[idx_v.at[i]], rows_v.at[b], gsem.at[b]
                ).wait()
                pltpu.sync_copy(
                    rows_v.at[b], out_hbm.at[pl.ds(base + i * _CHUNK, _CHUNK)]
                )

                @pl.when(g < n_groups - 1)
                def _():
                    gather(i + _NBUF, b)

            return carry

        lax.fori_loop(0, n_groups, group_body, 0)

    return emb


def kernel(input, weight):
    batch, seq = input.shape
    idx = input.reshape(-1, _CHUNK).astype(jnp.int32)
    out = _build(batch * seq)(idx, weight)
    return out.reshape(batch, seq, EMBEDDING_DIM)


# speedup vs baseline: 1.1964x; 1.0008x over previous
import functools

import jax
import jax.numpy as jnp
from jax import lax
from jax.experimental import pallas as pl
from jax.experimental.pallas import tpu as pltpu
from jax.experimental.pallas import tpu_sc as plsc

EMBEDDING_DIM = 64
_NUM_CORES = 2
_NUM_SUBCORES = 16
_NW = _NUM_CORES * _NUM_SUBCORES

_CHUNK = 128
_NBUF = 8
_AHEAD = 4


def _build(total_rows: int):
    assert total_rows % (_NW * _CHUNK) == 0
    per_worker = total_rows // _NW
    n_chunks = per_worker // _CHUNK
    assert n_chunks % _NBUF == 0 and n_chunks > _NBUF
    n_groups = n_chunks // _NBUF

    mesh = plsc.VectorSubcoreMesh(core_axis_name="c", subcore_axis_name="s")

    @functools.partial(
        pl.kernel,
        mesh=mesh,
        compiler_params=pltpu.CompilerParams(use_tc_tiling_on_sc=False),
        out_type=jax.ShapeDtypeStruct((total_rows, EMBEDDING_DIM), jnp.float32),
        scratch_types=[
            pltpu.VMEM((n_chunks, _CHUNK), jnp.int32),
            pltpu.VMEM((_NBUF, _CHUNK, EMBEDDING_DIM), jnp.float32),
            pltpu.SemaphoreType.DMA((_NBUF,)),
            pltpu.SemaphoreType.DMA((_NBUF,)),
        ],
    )
    def emb(idx_hbm, table_hbm, out_hbm, idx_v, rows_v, gsem, ssem):
        wid = lax.axis_index("s") * _NUM_CORES + lax.axis_index("c")
        base = wid * per_worker

        pltpu.sync_copy(idx_hbm.at[pl.ds(wid * n_chunks, n_chunks)], idx_v)

        def issue_gather(chunk, b):
            pltpu.async_copy(
                table_hbm.at[idx_v.at[chunk]], rows_v.at[b], gsem.at[b]
            )

        def wait_gather(chunk, b):
            pltpu.make_async_copy(
                table_hbm.at[idx_v.at[chunk]], rows_v.at[b], gsem.at[b]
            ).wait()

        def issue_store(chunk, b):
            pltpu.async_copy(
                rows_v.at[b],
                out_hbm.at[pl.ds(base + chunk * _CHUNK, _CHUNK)],
                ssem.at[b],
            )

        def wait_store(b):
            pltpu.make_async_copy(
                rows_v.at[b], out_hbm.at[pl.ds(base, _CHUNK)], ssem.at[b]
            ).wait()

        for b in range(_AHEAD):
            issue_gather(b, b)

        def group_body(g, carry):
            for b in range(_NBUF):
                j = g * _NBUF + b
                wait_gather(j, b)
                issue_store(j, b)
                c = (b + _AHEAD) % _NBUF

                @pl.when(jnp.logical_and(j + _AHEAD >= _NBUF,
                                         j + _AHEAD < n_chunks))
                def _():
                    wait_store(c)

                @pl.when(j + _AHEAD < n_chunks)
                def _():
                    issue_gather(j + _AHEAD, c)

            return carry

        lax.fori_loop(0, n_groups, group_body, 0)

        for k in range(n_chunks - _NBUF, n_chunks):
            wait_store(k % _NBUF)

    return emb


def kernel(input, weight):
    batch, seq = input.shape
    idx = input.reshape(-1, _CHUNK).astype(jnp.int32)
    out = _build(batch * seq)(idx, weight)
    return out.reshape(batch, seq, EMBEDDING_DIM)
